# baseline (speedup 1.0000x reference)
import jax
import jax.numpy as jnp
from jax.experimental import pallas as pl
from jax.experimental.pallas import tpu as pltpu

_FEAT = 64
_EMB = 128
_B2 = 4000


def _fused_kernel(x_ref, w1_ref, b1_ref, w2_ref, b2_ref, o_ref):
    xp = x_ref[...]
    h = jnp.dot(xp, w1_ref[...], preferred_element_type=jnp.float32) + b1_ref[...]
    h = jnp.maximum(h, 0.0)
    e = jnp.dot(h, w2_ref[...], preferred_element_type=jnp.float32) + b2_ref[...]
    ea = e[:, :_EMB]
    eb = e[:, _EMB:]
    sqa = jnp.sum(ea * ea, axis=-1, keepdims=True)
    sqb = jnp.sum(eb * eb, axis=-1, keepdims=True)
    o_ref[0] = jnp.where(sqa > 0.0, ea * jax.lax.rsqrt(sqa), 0.0)
    o_ref[1] = jnp.where(sqb > 0.0, eb * jax.lax.rsqrt(sqb), 0.0)


def kernel(x, W1, b1, W2, b2):
    n_rows = x.shape[0]
    n2 = n_rows // 2
    xw = jnp.concatenate([x[:n2], x[n2:]], axis=1)

    zero1 = jnp.zeros((_FEAT, _FEAT), dtype=jnp.float32)
    w1d = jnp.block([[W1, zero1], [zero1, W1]])
    zero2 = jnp.zeros((_FEAT, _EMB), dtype=jnp.float32)
    w2d = jnp.block([[W2, zero2], [zero2, W2]])
    b1d = jnp.concatenate([b1, b1]).reshape(1, 2 * _FEAT)
    b2d = jnp.concatenate([b2, b2]).reshape(1, 2 * _EMB)

    out3 = pl.pallas_call(
        _fused_kernel,
        grid=(n2 // _B2,),
        in_specs=[
            pl.BlockSpec((_B2, 2 * _FEAT), lambda i: (i, 0)),
            pl.BlockSpec((2 * _FEAT, 2 * _FEAT), lambda i: (0, 0)),
            pl.BlockSpec((1, 2 * _FEAT), lambda i: (0, 0)),
            pl.BlockSpec((2 * _FEAT, 2 * _EMB), lambda i: (0, 0)),
            pl.BlockSpec((1, 2 * _EMB), lambda i: (0, 0)),
        ],
        out_specs=pl.BlockSpec((2, _B2, _EMB), lambda i: (0, i, 0)),
        out_shape=jax.ShapeDtypeStruct((2, n2, _EMB), jnp.float32),
        compiler_params=pltpu.CompilerParams(
            dimension_semantics=("arbitrary",),
            vmem_limit_bytes=56 * 1024 * 1024,
        ),
    )(xw, w1d, b1d, w2d, b2d)
    return out3.reshape(n_rows, _EMB)


# speedup vs baseline: 1.5428x; 1.5428x over previous
import jax
import jax.numpy as jnp
from jax.experimental import pallas as pl
from jax.experimental.pallas import tpu as pltpu

_FEAT = 64
_EMB = 128
_BLOCK = 25000


def _fused_kernel(x_ref, w1_ref, b1_ref, w2_ref, b2_ref, o_ref):
    x = x_ref[...]
    h = jnp.dot(x, w1_ref[...], preferred_element_type=jnp.float32) + b1_ref[...]
    h = jnp.maximum(h, 0.0)
    e = jnp.dot(h, w2_ref[...], preferred_element_type=jnp.float32) + b2_ref[...]
    sq = jnp.sum(e * e, axis=-1, keepdims=True)
    inv = jax.lax.rsqrt(sq)
    o_ref[...] = jnp.where(sq > 0.0, e * inv, 0.0)


def kernel(x, W1, b1, W2, b2):
    n_rows = x.shape[0]
    grid = (n_rows // _BLOCK,)
    return pl.pallas_call(
        _fused_kernel,
        grid=grid,
        in_specs=[
            pl.BlockSpec((_BLOCK, _FEAT), lambda i: (i, 0)),
            pl.BlockSpec((_FEAT, _FEAT), lambda i: (0, 0)),
            pl.BlockSpec((1, _FEAT), lambda i: (0, 0)),
            pl.BlockSpec((_FEAT, _EMB), lambda i: (0, 0)),
            pl.BlockSpec((1, _EMB), lambda i: (0, 0)),
        ],
        out_specs=pl.BlockSpec((_BLOCK, _EMB), lambda i: (i, 0)),
        out_shape=jax.ShapeDtypeStruct((n_rows, _EMB), jnp.float32),
        compiler_params=pltpu.CompilerParams(
            dimension_semantics=("arbitrary",),
            vmem_limit_bytes=56 * 1024 * 1024,
        ),
    )(x, W1, b1.reshape(1, _FEAT), W2, b2.reshape(1, _EMB))


# speedup vs baseline: 1.5621x; 1.0125x over previous
import jax
import jax.numpy as jnp
from jax.experimental import pallas as pl
from jax.experimental.pallas import tpu as pltpu

_FEAT = 64
_EMB = 128
_BLOCK = 20000


def _fused_kernel(x_ref, w1_ref, b1_ref, w2_ref, b2_ref, o_ref):
    x = x_ref[...]
    h = jnp.dot(x, w1_ref[...], preferred_element_type=jnp.float32) + b1_ref[...]
    h = jnp.maximum(h, 0.0)
    e = jnp.dot(h, w2_ref[...], preferred_element_type=jnp.float32) + b2_ref[...]
    sq = jnp.sum(e * e, axis=-1, keepdims=True)
    inv = jax.lax.rsqrt(sq)
    o_ref[...] = jnp.where(sq > 0.0, e * inv, 0.0)


def kernel(x, W1, b1, W2, b2):
    n_rows = x.shape[0]
    grid = (n_rows // _BLOCK,)
    return pl.pallas_call(
        _fused_kernel,
        grid=grid,
        in_specs=[
            pl.BlockSpec((_BLOCK, _FEAT), lambda i: (i, 0)),
            pl.BlockSpec((_FEAT, _FEAT), lambda i: (0, 0)),
            pl.BlockSpec((1, _FEAT), lambda i: (0, 0)),
            pl.BlockSpec((_FEAT, _EMB), lambda i: (0, 0)),
            pl.BlockSpec((1, _EMB), lambda i: (0, 0)),
        ],
        out_specs=pl.BlockSpec((_BLOCK, _EMB), lambda i: (i, 0)),
        out_shape=jax.ShapeDtypeStruct((n_rows, _EMB), jnp.float32),
        compiler_params=pltpu.CompilerParams(
            dimension_semantics=("arbitrary",),
            vmem_limit_bytes=56 * 1024 * 1024,
        ),
    )(x, W1, b1.reshape(1, _FEAT), W2, b2.reshape(1, _EMB))
